# baseline (speedup 1.0000x reference)
import functools

import jax
import jax.numpy as jnp
from jax.experimental import pallas as pl
from jax.experimental.pallas import tpu as pltpu

_VMEM_LIMIT_BYTES = 64 * 1024 * 1024


def _gn_fused_kernel(x_ref, g_ref, b_ref, o_ref, *, eps, m):
    x = x_ref[...]
    srow = jnp.sum(x, axis=3, keepdims=True)
    ssrow = jnp.sum(x * x, axis=3, keepdims=True)
    s = jnp.sum(srow, axis=2, keepdims=True)
    ss = jnp.sum(ssrow, axis=2, keepdims=True)
    mean = s * (1.0 / m)
    var = jnp.maximum(ss - s * mean, 0.0) * (1.0 / (m - 1))
    inv = pl.reciprocal(jnp.sqrt(var) + jnp.float32(eps), approx=False)
    scale = g_ref[...] * inv
    bias = b_ref[...] - mean * scale
    o_ref[...] = x * scale + bias


def _group_norm_2d(x, gamma, beta, *, group_num, eps):
    n, c, h, w = x.shape
    g = group_num
    cg = c // g
    hw = h * w
    m = cg * hw

    x_r = x.reshape(n, g, cg, hw)
    gamma_r = gamma.reshape(1, g, cg, 1)
    beta_r = beta.reshape(1, g, cg, 1)

    gb = 16 if g % 16 == 0 else 1
    nb = 1

    fused = functools.partial(_gn_fused_kernel, eps=float(eps), m=m)
    out_r = pl.pallas_call(
        fused,
        out_shape=jax.ShapeDtypeStruct((n, g, cg, hw), x.dtype),
        grid=(g // gb, n // nb),
        in_specs=[
            pl.BlockSpec((nb, gb, cg, hw), lambda gi, ni: (ni, gi, 0, 0)),
            pl.BlockSpec((1, gb, cg, 1), lambda gi, ni: (0, gi, 0, 0)),
            pl.BlockSpec((1, gb, cg, 1), lambda gi, ni: (0, gi, 0, 0)),
        ],
        out_specs=pl.BlockSpec((nb, gb, cg, hw), lambda gi, ni: (ni, gi, 0, 0)),
        compiler_params=pltpu.CompilerParams(
            dimension_semantics=("parallel", "parallel"),
            vmem_limit_bytes=_VMEM_LIMIT_BYTES,
        ),
    )(x_r, gamma_r, beta_r)
    return out_r.reshape(n, c, h, w)


def kernel(x, gamma, beta):
    return _group_norm_2d(x, gamma, beta, group_num=32, eps=1e-10)


# speedup vs baseline: 1.2139x; 1.2139x over previous
import functools

import jax
import jax.numpy as jnp
from jax.experimental import pallas as pl
from jax.experimental.pallas import tpu as pltpu

_VMEM_LIMIT_BYTES = 64 * 1024 * 1024


def _gn_native_kernel(x_ref, g_ref, b_ref, o_ref, *, eps, m, cg):
    gi = pl.program_id(0)
    x = x_ref[...]
    s = jnp.sum(x, axis=3, keepdims=True)
    s = jnp.sum(s, axis=2, keepdims=True)
    s = jnp.sum(s, axis=1, keepdims=True)
    ss = jnp.sum(x * x, axis=3, keepdims=True)
    ss = jnp.sum(ss, axis=2, keepdims=True)
    ss = jnp.sum(ss, axis=1, keepdims=True)
    mean = s * (1.0 / m)
    var = jnp.maximum(ss - s * mean, 0.0) * (1.0 / (m - 1))
    inv = pl.reciprocal(jnp.sqrt(var) + jnp.float32(eps), approx=False)
    for c in range(cg):
        gc = g_ref[gi * cg + c]
        bc = b_ref[gi * cg + c]
        sc = gc * inv
        off = bc - mean * sc
        o_ref[:, c:c + 1, :, :] = x[:, c:c + 1, :, :] * sc + off


def _group_norm_2d(x, gamma, beta, *, group_num, eps):
    n, c, h, w = x.shape
    g = group_num
    cg = c // g
    m = cg * h * w

    gamma_s = gamma.reshape(c)
    beta_s = beta.reshape(c)

    fused = functools.partial(_gn_native_kernel, eps=float(eps), m=m, cg=cg)
    out = pl.pallas_call(
        fused,
        out_shape=jax.ShapeDtypeStruct((n, c, h, w), x.dtype),
        grid=(g,),
        in_specs=[
            pl.BlockSpec((n, cg, h, w), lambda gi: (0, gi, 0, 0)),
            pl.BlockSpec(memory_space=pltpu.SMEM),
            pl.BlockSpec(memory_space=pltpu.SMEM),
        ],
        out_specs=pl.BlockSpec((n, cg, h, w), lambda gi: (0, gi, 0, 0)),
        compiler_params=pltpu.CompilerParams(
            dimension_semantics=("parallel",),
            vmem_limit_bytes=_VMEM_LIMIT_BYTES,
        ),
    )(x, gamma_s, beta_s)
    return out


def kernel(x, gamma, beta):
    return _group_norm_2d(x, gamma, beta, group_num=32, eps=1e-10)


# speedup vs baseline: 8.3270x; 6.8597x over previous
import functools

import jax
import jax.numpy as jnp
from jax import lax
from jax.experimental import pallas as pl
from jax.experimental.pallas import tpu as pltpu

_VMEM_LIMIT_BYTES = 64 * 1024 * 1024


def _gn_nhwc_kernel(x_ref, g_ref, b_ref, o_ref, *, eps, m, cg):
    c = x_ref.shape[3]
    x = x_ref[...]
    xx = x * x
    s = jnp.sum(x, axis=1, keepdims=True)
    s = jnp.sum(s, axis=2, keepdims=True)
    ss = jnp.sum(xx, axis=1, keepdims=True)
    ss = jnp.sum(ss, axis=2, keepdims=True)

    v = jnp.concatenate([s.reshape(1, c), ss.reshape(1, c)], axis=0)
    ci = lax.broadcasted_iota(jnp.int32, (c, c), 0) // cg
    cj = lax.broadcasted_iota(jnp.int32, (c, c), 1) // cg
    mask = (ci == cj).astype(jnp.float32)
    gv = jnp.dot(v, mask, preferred_element_type=jnp.float32)

    gs = gv[0:1, :]
    gss = gv[1:2, :]
    mean = gs * (1.0 / m)
    var = jnp.maximum(gss - gs * mean, 0.0) * (1.0 / (m - 1))
    inv = pl.reciprocal(jnp.sqrt(var) + jnp.float32(eps), approx=False)
    scale = g_ref[...] * inv
    bias = b_ref[...] - mean * scale
    o_ref[...] = x * scale.reshape(1, 1, 1, c) + bias.reshape(1, 1, 1, c)


def _group_norm_2d(x, gamma, beta, *, group_num, eps):
    n, c, h, w = x.shape
    g = group_num
    cg = c // g
    m = cg * h * w

    x_t = jnp.transpose(x, (0, 2, 3, 1))
    gamma_r = gamma.reshape(1, c)
    beta_r = beta.reshape(1, c)

    fused = functools.partial(_gn_nhwc_kernel, eps=float(eps), m=m, cg=cg)
    out_t = pl.pallas_call(
        fused,
        out_shape=jax.ShapeDtypeStruct((n, h, w, c), x.dtype),
        grid=(n,),
        in_specs=[
            pl.BlockSpec((1, h, w, c), lambda ni: (ni, 0, 0, 0)),
            pl.BlockSpec((1, c), lambda ni: (0, 0)),
            pl.BlockSpec((1, c), lambda ni: (0, 0)),
        ],
        out_specs=pl.BlockSpec((1, h, w, c), lambda ni: (ni, 0, 0, 0)),
        compiler_params=pltpu.CompilerParams(
            dimension_semantics=("parallel",),
            vmem_limit_bytes=_VMEM_LIMIT_BYTES,
        ),
    )(x_t, gamma_r, beta_r)
    return jnp.transpose(out_t, (0, 3, 1, 2))


def kernel(x, gamma, beta):
    return _group_norm_2d(x, gamma, beta, group_num=32, eps=1e-10)


# speedup vs baseline: 8.6244x; 1.0357x over previous
import functools

import jax
import jax.numpy as jnp
from jax import lax
from jax.experimental import pallas as pl
from jax.experimental.pallas import tpu as pltpu

_VMEM_LIMIT_BYTES = 64 * 1024 * 1024


def _gn_nhwc_kernel(x_ref, g_ref, b_ref, o_ref, *, eps, m, cg):
    nb, _, _, c = x_ref.shape
    x = x_ref[...]
    xx = x * x
    s = jnp.sum(x, axis=1, keepdims=True)
    s = jnp.sum(s, axis=2, keepdims=True)
    ss = jnp.sum(xx, axis=1, keepdims=True)
    ss = jnp.sum(ss, axis=2, keepdims=True)

    v = jnp.concatenate([s.reshape(nb, c), ss.reshape(nb, c)], axis=0)
    ci = lax.broadcasted_iota(jnp.int32, (c, c), 0) // cg
    cj = lax.broadcasted_iota(jnp.int32, (c, c), 1) // cg
    mask = (ci == cj).astype(jnp.float32)
    gv = jnp.dot(v, mask, preferred_element_type=jnp.float32)

    gs = gv[0:nb, :]
    gss = gv[nb:2 * nb, :]
    mean = gs * (1.0 / m)
    var = jnp.maximum(gss - gs * mean, 0.0) * (1.0 / (m - 1))
    inv = pl.reciprocal(jnp.sqrt(var) + jnp.float32(eps), approx=False)
    scale = g_ref[...] * inv
    bias = b_ref[...] - mean * scale
    o_ref[...] = (x * scale.reshape(nb, 1, 1, c)
                  + bias.reshape(nb, 1, 1, c))


def _group_norm_2d(x, gamma, beta, *, group_num, eps):
    n, c, h, w = x.shape
    g = group_num
    cg = c // g
    m = cg * h * w

    x_t = jnp.transpose(x, (0, 2, 3, 1))
    gamma_r = gamma.reshape(1, c)
    beta_r = beta.reshape(1, c)

    nb = 2
    fused = functools.partial(_gn_nhwc_kernel, eps=float(eps), m=m, cg=cg)
    out_t = pl.pallas_call(
        fused,
        out_shape=jax.ShapeDtypeStruct((n, h, w, c), x.dtype),
        grid=(n // nb,),
        in_specs=[
            pl.BlockSpec((nb, h, w, c), lambda ni: (ni, 0, 0, 0)),
            pl.BlockSpec((1, c), lambda ni: (0, 0)),
            pl.BlockSpec((1, c), lambda ni: (0, 0)),
        ],
        out_specs=pl.BlockSpec((nb, h, w, c), lambda ni: (ni, 0, 0, 0)),
        compiler_params=pltpu.CompilerParams(
            dimension_semantics=("parallel",),
            vmem_limit_bytes=_VMEM_LIMIT_BYTES,
        ),
    )(x_t, gamma_r, beta_r)
    return jnp.transpose(out_t, (0, 3, 1, 2))


def kernel(x, gamma, beta):
    return _group_norm_2d(x, gamma, beta, group_num=32, eps=1e-10)
